# Initial kernel scaffold; baseline (speedup 1.0000x reference)
import jax
import jax.numpy as jnp
from jax.experimental import pallas as pl


def kernel(atom_pad_mask, atom_mono_idx, token_bonds, atom_to_token):
    raise NotImplementedError("write your pallas kernel here")


# speedup vs baseline: 236.4581x; 236.4581x over previous
import jax
import jax.numpy as jnp
from jax import lax
from jax.experimental import pallas as pl

_INF = 1024.0


def _fw_body(mono_col_ref, mono_row_ref, tb_ref, a2t_ref, out_ref):
    N = a2t_ref.shape[1]
    T = a2t_ref.shape[2]
    x = a2t_ref[0]
    tb = tb_ref[0]
    mono_col = mono_col_ref[0]
    mono_row = mono_row_ref[0]

    lane = lax.broadcasted_iota(jnp.int32, (N, T), 1)
    m = jnp.max(x, axis=1, keepdims=True)
    idx = jnp.min(jnp.where(x == m, lane, T), axis=1, keepdims=True)
    P = (lane == idx).astype(jnp.float32)

    gly_col = (mono_col != -1)
    Pg = P * gly_col.astype(jnp.float32)
    occ = jnp.max(Pg, axis=0, keepdims=True)

    D0 = jnp.where((tb > 0.0) & (occ > 0.5), 1.0, _INF)

    li = lax.broadcasted_iota(jnp.int32, (T, T), 1)
    si = lax.broadcasted_iota(jnp.int32, (T, T), 0)

    def body(k, D):
        col = jnp.min(jnp.where(li == k, D, 4.0 * _INF), axis=1, keepdims=True)
        row = jnp.min(jnp.where(si == k, D, 4.0 * _INF), axis=0, keepdims=True)
        return jnp.minimum(D, col + row)

    D = lax.fori_loop(0, T, body, D0)

    R = lax.dot_general(P, D, (((1,), (0,)), ((), ())),
                        preferred_element_type=jnp.float32)
    O = lax.dot_general(R, P, (((1,), (1,)), ((), ())),
                        preferred_element_type=jnp.float32)

    gly_row = (mono_row != -1)
    li2 = lax.broadcasted_iota(jnp.int32, (N, N), 1)
    si2 = lax.broadcasted_iota(jnp.int32, (N, N), 0)
    vals = O.astype(jnp.int32)
    vals = jnp.where(O > 500.0, -1, vals)
    vals = jnp.where(gly_col & gly_row, vals, -1)
    vals = jnp.where(li2 == si2, 0, vals)
    out_ref[0] = vals


def kernel(atom_pad_mask, atom_mono_idx, token_bonds, atom_to_token):
    B, N = atom_pad_mask.shape
    T = token_bonds.shape[1]
    tb = jnp.squeeze(token_bonds, -1)
    mono_col = atom_mono_idx.reshape(B, N, 1)
    mono_row = atom_mono_idx.reshape(B, 1, N)
    out = pl.pallas_call(
        _fw_body,
        grid=(B,),
        in_specs=[
            pl.BlockSpec((1, N, 1), lambda b: (b, 0, 0)),
            pl.BlockSpec((1, 1, N), lambda b: (b, 0, 0)),
            pl.BlockSpec((1, T, T), lambda b: (b, 0, 0)),
            pl.BlockSpec((1, N, T), lambda b: (b, 0, 0)),
        ],
        out_specs=pl.BlockSpec((1, N, N), lambda b: (b, 0, 0)),
        out_shape=jax.ShapeDtypeStruct((B, N, N), jnp.int32),
    )(mono_col, mono_row, tb, atom_to_token)
    return out


# speedup vs baseline: 297.5323x; 1.2583x over previous
import jax
import jax.numpy as jnp
from jax import lax
from jax.experimental import pallas as pl

_INF = 1024.0


def _fw_body(mono_col_ref, mono_row_ref, tb_ref, a2t_ref, out_ref):
    N = a2t_ref.shape[1]
    T = a2t_ref.shape[2]
    x = a2t_ref[0]
    tb = tb_ref[0]
    mono_col = mono_col_ref[0]
    mono_row = mono_row_ref[0]

    lane = lax.broadcasted_iota(jnp.int32, (N, T), 1)
    m = jnp.max(x, axis=1, keepdims=True)
    idx = jnp.min(jnp.where(x == m, lane, T), axis=1, keepdims=True)
    P = (lane == idx).astype(jnp.float32)

    gly_col = (mono_col != -1)
    Pg = P * gly_col.astype(jnp.float32)
    occ = jnp.max(Pg, axis=0, keepdims=True)

    D0 = jnp.where((tb > 0.0) & (occ > 0.5), 1.0, _INF)

    D = D0
    for k in range(T):
        col = lax.slice(D, (0, k), (T, k + 1))
        row = lax.slice(D, (k, 0), (k + 1, T))
        D = jnp.minimum(D, col + row)

    R = lax.dot_general(P, D, (((1,), (0,)), ((), ())),
                        preferred_element_type=jnp.float32)
    O = lax.dot_general(R, P, (((1,), (1,)), ((), ())),
                        preferred_element_type=jnp.float32)

    gly_row = (mono_row != -1)
    li2 = lax.broadcasted_iota(jnp.int32, (N, N), 1)
    si2 = lax.broadcasted_iota(jnp.int32, (N, N), 0)
    vals = O.astype(jnp.int32)
    vals = jnp.where(O > 500.0, -1, vals)
    vals = jnp.where(gly_col & gly_row, vals, -1)
    vals = jnp.where(li2 == si2, 0, vals)
    out_ref[0] = vals


def kernel(atom_pad_mask, atom_mono_idx, token_bonds, atom_to_token):
    B, N = atom_pad_mask.shape
    T = token_bonds.shape[1]
    tb = jnp.squeeze(token_bonds, -1)
    mono_col = atom_mono_idx.reshape(B, N, 1)
    mono_row = atom_mono_idx.reshape(B, 1, N)
    out = pl.pallas_call(
        _fw_body,
        grid=(B,),
        in_specs=[
            pl.BlockSpec((1, N, 1), lambda b: (b, 0, 0)),
            pl.BlockSpec((1, 1, N), lambda b: (b, 0, 0)),
            pl.BlockSpec((1, T, T), lambda b: (b, 0, 0)),
            pl.BlockSpec((1, N, T), lambda b: (b, 0, 0)),
        ],
        out_specs=pl.BlockSpec((1, N, N), lambda b: (b, 0, 0)),
        out_shape=jax.ShapeDtypeStruct((B, N, N), jnp.int32),
    )(mono_col, mono_row, tb, atom_to_token)
    return out


# speedup vs baseline: 349.5951x; 1.1750x over previous
import jax
import jax.numpy as jnp
from jax import lax
from jax.experimental import pallas as pl

_INF = 1024.0
_BK = 8


def _fw_closed(D0, T):
    D = D0
    for k0 in range(0, T, _BK):
        R = lax.slice(D, (k0, 0), (k0 + _BK, T))
        for k in range(_BK):
            col = lax.slice(R, (0, k0 + k), (_BK, k0 + k + 1))
            row = lax.slice(R, (k, 0), (k + 1, T))
            R = jnp.minimum(R, col + row)
        terms = []
        for k in range(_BK):
            colf = lax.slice(D, (0, k0 + k), (T, k0 + k + 1))
            rowf = lax.slice(R, (k, 0), (k + 1, T))
            terms.append(colf + rowf)
        while len(terms) > 1:
            terms = [jnp.minimum(terms[i], terms[i + 1])
                     for i in range(0, len(terms), 2)]
        D = jnp.minimum(D, terms[0])
    return D


def _batch_distances(x, tb, mono_col, N, T):
    lane = lax.broadcasted_iota(jnp.int32, (N, T), 1)
    m = jnp.max(x, axis=1, keepdims=True)
    idx = jnp.min(jnp.where(x == m, lane, T), axis=1, keepdims=True)
    P = (lane == idx).astype(jnp.float32)

    gly_col = (mono_col != -1)
    Pg = P * gly_col.astype(jnp.float32)
    occ = jnp.max(Pg, axis=0, keepdims=True)

    D0 = jnp.where((tb > 0.0) & (occ > 0.5), 1.0, _INF)
    D = _fw_closed(D0, T)
    return P, D, gly_col


def _expand(P, D, gly_col, gly_row, N):
    R = lax.dot_general(P, D, (((1,), (0,)), ((), ())),
                        preferred_element_type=jnp.float32)
    O = lax.dot_general(R, P, (((1,), (1,)), ((), ())),
                        preferred_element_type=jnp.float32)
    li2 = lax.broadcasted_iota(jnp.int32, (N, N), 1)
    si2 = lax.broadcasted_iota(jnp.int32, (N, N), 0)
    vals = O.astype(jnp.int32)
    vals = jnp.where(O > 500.0, -1, vals)
    vals = jnp.where(gly_col & gly_row, vals, -1)
    vals = jnp.where(li2 == si2, 0, vals)
    return vals


def _fw_body(mono_col_ref, mono_row_ref, tb_ref, a2t_ref, out_ref):
    B = a2t_ref.shape[0]
    N = a2t_ref.shape[1]
    T = a2t_ref.shape[2]
    per_batch = [
        _batch_distances(a2t_ref[b], tb_ref[b], mono_col_ref[b], N, T)
        for b in range(B)
    ]
    for b in range(B):
        P, D, gly_col = per_batch[b]
        out_ref[b] = _expand(P, D, gly_col, mono_row_ref[b] != -1, N)


def kernel(atom_pad_mask, atom_mono_idx, token_bonds, atom_to_token):
    B, N = atom_pad_mask.shape
    T = token_bonds.shape[1]
    tb = jnp.squeeze(token_bonds, -1)
    mono_col = atom_mono_idx.reshape(B, N, 1)
    mono_row = atom_mono_idx.reshape(B, 1, N)
    out = pl.pallas_call(
        _fw_body,
        out_shape=jax.ShapeDtypeStruct((B, N, N), jnp.int32),
    )(mono_col, mono_row, tb, atom_to_token)
    return out
